# baseline (speedup 1.0000x reference)
import functools

import jax
import jax.numpy as jnp
from jax import lax
from jax.experimental import pallas as pl
from jax.experimental.pallas import tpu as pltpu
from jax.experimental.pallas import tpu_sc as plsc

_B = 64
_IN_DIM = 8
_IMG_DIM = 1152
_TOK = 64
_EMB = 2048
_K = 512
_NTOK = 576
_HALF = 1024

_KB = 32
_PCH = 32


def _codesum_body(style_ref, out_ref):
    x = style_ref[...]
    cs = jnp.sum(x, axis=1)
    norm = jnp.sqrt(jnp.sum(cs * cs, axis=1, keepdims=True))
    out_ref[...] = cs / jnp.maximum(norm, _TOK * 1e-6)


def _query_body(isum_ref, emo_ref, wemo_ref, bemo_ref, wimg_ref, bimg_ref,
                q_ref):
    pooled = isum_ref[...] * (1.0 / _NTOK)
    q_img = lax.dot_general(
        pooled, wimg_ref[...], (((1,), (1,)), ((), ())),
        preferred_element_type=jnp.float32,
        precision=lax.Precision.HIGHEST) + bimg_ref[...]
    q_emo = lax.dot_general(
        emo_ref[...], wemo_ref[...], (((1,), (1,)), ((), ())),
        preferred_element_type=jnp.float32,
        precision=lax.Precision.HIGHEST) + bemo_ref[...]
    q = jnp.concatenate([q_emo, q_img], axis=1)
    q_ref[...] = q - jnp.mean(q, axis=1, keepdims=True)


def _select_body(cn_ref, q_ref, idx_ref, rows_ref):
    s = lax.dot_general(
        cn_ref[...], q_ref[...], (((1,), (1,)), ((), ())),
        preferred_element_type=jnp.float32,
        precision=lax.Precision.HIGHEST)
    m = jnp.max(s, axis=0, keepdims=True)
    kiota = lax.broadcasted_iota(jnp.int32, (_K, _B), 0)
    cand = jnp.where(s == m, kiota, _K)
    idx = jnp.min(cand, axis=0, keepdims=True)
    idx_ref[...] = idx
    eye = (lax.broadcasted_iota(jnp.int32, (_B, _B), 0) ==
           lax.broadcasted_iota(jnp.int32, (_B, _B), 1)).astype(jnp.float32)
    idx_col = lax.dot_general(
        eye, idx.astype(jnp.float32), (((1,), (1,)), ((), ())),
        preferred_element_type=jnp.float32,
        precision=lax.Precision.HIGHEST)
    tiota = lax.broadcasted_iota(jnp.int32, (_B, _TOK), 1)
    rows_ref[...] = idx_col.astype(jnp.int32) * _TOK + tiota


@functools.lru_cache(maxsize=1)
def _make_sc_pool():
    NC, NS = 2, 16
    b_per_w = _B // (NC * NS)
    n_ch = _NTOK // _PCH
    n_vec = _IMG_DIM // 16

    @functools.partial(
        pl.kernel,
        out_type=jax.ShapeDtypeStruct((_B, _IMG_DIM), jnp.float32),
        mesh=plsc.VectorSubcoreMesh(core_axis_name="c", subcore_axis_name="s"),
        scratch_types=[
            pltpu.VMEM((2, _PCH, _IMG_DIM), jnp.float32),
            pltpu.VMEM((b_per_w, _IMG_DIM), jnp.float32),
            pltpu.SemaphoreType.DMA,
            pltpu.SemaphoreType.DMA,
        ],
    )
    def pool_kernel(img_hbm, out_hbm, buf, acc, s0, s1):
        c_id = lax.axis_index("c")
        s_id = lax.axis_index("s")
        sem = (s0, s1)
        gb0 = (c_id * NS + s_id) * b_per_w

        zero = jnp.zeros((16,), jnp.float32)
        for lb in range(b_per_w):
            for d in range(n_vec):
                acc[lb, pl.ds(16 * d, 16)] = zero

        def add_chunk(jb, lb):
            def body(t, carry):
                for d in range(n_vec):
                    v = buf[jb, t, pl.ds(16 * d, 16)]
                    plsc.addupdate(acc.at[lb, pl.ds(16 * d, 16)], v)
                return carry
            lax.fori_loop(0, _PCH, body, 0)

        def drain(jb):
            pltpu.make_async_copy(
                img_hbm.at[0, pl.ds(0, _PCH)], buf.at[jb], sem[jb]).wait()

        for lb in range(b_per_w):
            gb = gb0 + lb
            pltpu.async_copy(
                img_hbm.at[gb, pl.ds(0, _PCH)], buf.at[0], sem[0])
            pltpu.async_copy(
                img_hbm.at[gb, pl.ds(_PCH, _PCH)], buf.at[1], sem[1])

            def pair_body(i, carry):
                g = i * 2
                for jb in range(2):
                    drain(jb)
                    add_chunk(jb, lb)
                    nxt = g + 2 + jb

                    @pl.when(nxt < n_ch)
                    def _start():
                        pltpu.async_copy(
                            img_hbm.at[gb, pl.ds(nxt * _PCH, _PCH)],
                            buf.at[jb], sem[jb])
                return carry

            lax.fori_loop(0, n_ch // 2, pair_body, 0)

        pltpu.sync_copy(acc, out_hbm.at[pl.ds(gb0, b_per_w)])

    return pool_kernel


@functools.lru_cache(maxsize=1)
def _make_sc_gather():
    NC, NS = 2, 16
    NW = NC * NS
    NROWS = _B * _TOK
    CH = 16
    rows_per_w = NROWS // NW
    n_ch = rows_per_w // CH

    @functools.partial(
        pl.kernel,
        out_type=jax.ShapeDtypeStruct((NROWS, _EMB), jnp.float32),
        mesh=plsc.VectorSubcoreMesh(core_axis_name="c", subcore_axis_name="s"),
        scratch_types=[
            pltpu.VMEM((rows_per_w,), jnp.int32),
            pltpu.VMEM((2, CH, _EMB), jnp.float32),
            pltpu.SemaphoreType.DMA,
            pltpu.SemaphoreType.DMA,
            pltpu.SemaphoreType.DMA,
            pltpu.SemaphoreType.DMA,
        ],
    )
    def gather_kernel(table_hbm, rows_hbm, out_hbm, idxall, buf, g0, g1, w0,
                      w1):
        gsem = (g0, g1)
        wsem = (w0, w1)
        wid = lax.axis_index("s") * NC + lax.axis_index("c")
        wbase = wid * rows_per_w
        pltpu.sync_copy(rows_hbm.at[pl.ds(wbase, rows_per_w)], idxall)

        def gather(j):
            return pltpu.async_copy(
                table_hbm.at[idxall.at[pl.ds(j * CH, CH)]],
                buf.at[j % 2], gsem[j % 2])

        def scatter(j):
            return pltpu.async_copy(
                buf.at[j % 2], out_hbm.at[pl.ds(wbase + j * CH, CH)],
                wsem[j % 2])

        gh = [None] * n_ch
        sh = [None] * n_ch
        gh[0] = gather(0)
        for j in range(n_ch):
            if j + 1 < n_ch:
                if j >= 1:
                    sh[j - 1].wait()
                gh[j + 1] = gather(j + 1)
            gh[j].wait()
            sh[j] = scatter(j)
        sh[n_ch - 2].wait()
        sh[n_ch - 1].wait()

    return gather_kernel


def kernel(emo_vec, image_tokens, W_emo, b_emo, W_img, b_img, style_dict):
    img_sum = _make_sc_pool()(image_tokens)

    code_norm = pl.pallas_call(
        _codesum_body,
        grid=(_K // _KB,),
        in_specs=[pl.BlockSpec((_KB, _TOK, _EMB), lambda g: (g, 0, 0))],
        out_specs=pl.BlockSpec((_KB, _EMB), lambda g: (g, 0)),
        out_shape=jax.ShapeDtypeStruct((_K, _EMB), jnp.float32),
    )(style_dict)

    qc = pl.pallas_call(
        _query_body,
        in_specs=[
            pl.BlockSpec((_B, _IMG_DIM), lambda: (0, 0)),
            pl.BlockSpec((_B, _IN_DIM), lambda: (0, 0)),
            pl.BlockSpec((_HALF, _IN_DIM), lambda: (0, 0)),
            pl.BlockSpec((1, _HALF), lambda: (0, 0)),
            pl.BlockSpec((_HALF, _IMG_DIM), lambda: (0, 0)),
            pl.BlockSpec((1, _HALF), lambda: (0, 0)),
        ],
        out_specs=pl.BlockSpec((_B, _EMB), lambda: (0, 0)),
        out_shape=jax.ShapeDtypeStruct((_B, _EMB), jnp.float32),
    )(img_sum, emo_vec, W_emo, b_emo.reshape(1, _HALF), W_img,
      b_img.reshape(1, _HALF))

    idx2d, rows2d = pl.pallas_call(
        _select_body,
        in_specs=[pl.BlockSpec((_K, _EMB), lambda: (0, 0)),
                  pl.BlockSpec((_B, _EMB), lambda: (0, 0))],
        out_specs=[pl.BlockSpec((1, _B), lambda: (0, 0)),
                   pl.BlockSpec((_B, _TOK), lambda: (0, 0))],
        out_shape=[jax.ShapeDtypeStruct((1, _B), jnp.int32),
                   jax.ShapeDtypeStruct((_B, _TOK), jnp.int32)],
    )(code_norm, qc)

    indices = idx2d.reshape(_B)
    table = style_dict.reshape(_K * _TOK, _EMB)
    rows = rows2d.reshape(_B * _TOK)
    style = _make_sc_gather()(table, rows).reshape(_B, _TOK, _EMB)
    return style, indices


# speedup vs baseline: 1.8035x; 1.8035x over previous
import functools

import jax
import jax.numpy as jnp
from jax import lax
from jax.experimental import pallas as pl
from jax.experimental.pallas import tpu as pltpu
from jax.experimental.pallas import tpu_sc as plsc

_B = 64
_IN_DIM = 8
_IMG_DIM = 1152
_TOK = 64
_EMB = 2048
_K = 512
_NTOK = 576
_HALF = 1024

_TB = 48
_KB = 32


def _query_body(img_ref, emo_ref, wemo_ref, bemo_ref, wimg_ref, bimg_ref,
                q_ref, acc_ref):
    g = pl.program_id(0)

    @pl.when(g == 0)
    def _init():
        acc_ref[...] = jnp.zeros_like(acc_ref)

    s = img_ref[:, 0:8, :]
    for t in range(1, _TB // 8):
        s = s + img_ref[:, 8 * t:8 * (t + 1), :]
    acc_ref[...] += s

    @pl.when(g == pl.num_programs(0) - 1)
    def _epilogue():
        pooled = jnp.sum(acc_ref[...], axis=1) * (1.0 / _NTOK)
        q_img = lax.dot_general(
            pooled, wimg_ref[...], (((1,), (1,)), ((), ())),
            preferred_element_type=jnp.float32,
            precision=lax.Precision.HIGHEST) + bimg_ref[...]
        q_emo = lax.dot_general(
            emo_ref[...], wemo_ref[...], (((1,), (1,)), ((), ())),
            preferred_element_type=jnp.float32,
            precision=lax.Precision.HIGHEST) + bemo_ref[...]
        q = jnp.concatenate([q_emo, q_img], axis=1)
        q_ref[...] = q - jnp.mean(q, axis=1, keepdims=True)


def _select_body(style_ref, q_ref, idx_ref, rows_ref, sacc_ref):
    g = pl.program_id(0)
    x = style_ref[...]
    cs = jnp.sum(x, axis=1)
    norm = jnp.sqrt(jnp.sum(cs * cs, axis=1, keepdims=True))
    denom = jnp.maximum(norm, _TOK * 1e-6)
    sblk = lax.dot_general(
        cs, q_ref[...], (((1,), (1,)), ((), ())),
        preferred_element_type=jnp.float32,
        precision=lax.Precision.HIGHEST)
    sacc_ref[g] = sblk / denom

    @pl.when(g == pl.num_programs(0) - 1)
    def _argmax():
        _argmax_epilogue(sacc_ref, idx_ref, rows_ref)


def _argmax_epilogue(sacc_ref, idx_ref, rows_ref):
    s = sacc_ref[...].reshape(_K, _B)
    m = jnp.max(s, axis=0, keepdims=True)
    kiota = lax.broadcasted_iota(jnp.int32, (_K, _B), 0)
    cand = jnp.where(s == m, kiota, _K)
    idx = jnp.min(cand, axis=0, keepdims=True)
    idx_ref[...] = idx
    eye = (lax.broadcasted_iota(jnp.int32, (_B, _B), 0) ==
           lax.broadcasted_iota(jnp.int32, (_B, _B), 1)).astype(jnp.float32)
    idx_col = lax.dot_general(
        eye, idx.astype(jnp.float32), (((1,), (1,)), ((), ())),
        preferred_element_type=jnp.float32,
        precision=lax.Precision.HIGHEST)
    tiota = lax.broadcasted_iota(jnp.int32, (_B, _TOK), 1)
    rows_ref[...] = idx_col.astype(jnp.int32) * _TOK + tiota


@functools.lru_cache(maxsize=1)
def _make_sc_gather():
    NC, NS = 2, 16
    NW = NC * NS
    NROWS = _B * _TOK
    CH = 16
    rows_per_w = NROWS // NW
    n_ch = rows_per_w // CH

    @functools.partial(
        pl.kernel,
        out_type=jax.ShapeDtypeStruct((NROWS, _EMB), jnp.float32),
        mesh=plsc.VectorSubcoreMesh(core_axis_name="c", subcore_axis_name="s"),
        scratch_types=[
            pltpu.VMEM((rows_per_w,), jnp.int32),
            pltpu.VMEM((2, CH, _EMB), jnp.float32),
            pltpu.SemaphoreType.DMA,
            pltpu.SemaphoreType.DMA,
            pltpu.SemaphoreType.DMA,
            pltpu.SemaphoreType.DMA,
        ],
    )
    def gather_kernel(table_hbm, rows_hbm, out_hbm, idxall, buf, g0, g1, w0,
                      w1):
        gsem = (g0, g1)
        wsem = (w0, w1)
        wid = lax.axis_index("s") * NC + lax.axis_index("c")
        wbase = wid * rows_per_w
        pltpu.sync_copy(rows_hbm.at[pl.ds(wbase, rows_per_w)], idxall)

        def gather(j):
            return pltpu.async_copy(
                table_hbm.at[idxall.at[pl.ds(j * CH, CH)]],
                buf.at[j % 2], gsem[j % 2])

        def scatter(j):
            return pltpu.async_copy(
                buf.at[j % 2], out_hbm.at[pl.ds(wbase + j * CH, CH)],
                wsem[j % 2])

        gh = [None] * n_ch
        sh = [None] * n_ch
        gh[0] = gather(0)
        for j in range(n_ch):
            if j + 1 < n_ch:
                if j >= 1:
                    sh[j - 1].wait()
                gh[j + 1] = gather(j + 1)
            gh[j].wait()
            sh[j] = scatter(j)
        sh[n_ch - 2].wait()
        sh[n_ch - 1].wait()

    return gather_kernel


def kernel(emo_vec, image_tokens, W_emo, b_emo, W_img, b_img, style_dict):
    qc = pl.pallas_call(
        _query_body,
        grid=(_NTOK // _TB,),
        in_specs=[
            pl.BlockSpec((_B, _TB, _IMG_DIM), lambda g: (0, g, 0)),
            pl.BlockSpec((_B, _IN_DIM), lambda g: (0, 0)),
            pl.BlockSpec((_HALF, _IN_DIM), lambda g: (0, 0)),
            pl.BlockSpec((1, _HALF), lambda g: (0, 0)),
            pl.BlockSpec((_HALF, _IMG_DIM), lambda g: (0, 0)),
            pl.BlockSpec((1, _HALF), lambda g: (0, 0)),
        ],
        out_specs=pl.BlockSpec((_B, _EMB), lambda g: (0, 0)),
        out_shape=jax.ShapeDtypeStruct((_B, _EMB), jnp.float32),
        scratch_shapes=[pltpu.VMEM((_B, 8, _IMG_DIM), jnp.float32)],
    )(image_tokens, emo_vec, W_emo, b_emo.reshape(1, _HALF), W_img,
      b_img.reshape(1, _HALF))

    idx2d, rows2d = pl.pallas_call(
        _select_body,
        grid=(_K // _KB,),
        in_specs=[
            pl.BlockSpec((_KB, _TOK, _EMB), lambda g: (g, 0, 0)),
            pl.BlockSpec((_B, _EMB), lambda g: (0, 0)),
        ],
        out_specs=[pl.BlockSpec((1, _B), lambda g: (0, 0)),
                   pl.BlockSpec((_B, _TOK), lambda g: (0, 0))],
        out_shape=[jax.ShapeDtypeStruct((1, _B), jnp.int32),
                   jax.ShapeDtypeStruct((_B, _TOK), jnp.int32)],
        scratch_shapes=[pltpu.VMEM((_K // _KB, _KB, _B), jnp.float32)],
    )(style_dict, qc)

    indices = idx2d.reshape(_B)
    table = style_dict.reshape(_K * _TOK, _EMB)
    rows = rows2d.reshape(_B * _TOK)
    style = _make_sc_gather()(table, rows).reshape(_B, _TOK, _EMB)
    return style, indices


# speedup vs baseline: 1.8115x; 1.0045x over previous
import functools

import jax
import jax.numpy as jnp
from jax import lax
from jax.experimental import pallas as pl
from jax.experimental.pallas import tpu as pltpu
from jax.experimental.pallas import tpu_sc as plsc

_B = 64
_IN_DIM = 8
_IMG_DIM = 1152
_TOK = 64
_EMB = 2048
_K = 512
_NTOK = 576
_HALF = 1024

_TB = 48
_KB = 32


def _query_body(img_ref, emo_ref, wemo_ref, bemo_ref, wimg_ref, bimg_ref,
                q_ref, acc_ref):
    g = pl.program_id(0)

    @pl.when(g == 0)
    def _init():
        acc_ref[...] = jnp.zeros_like(acc_ref)

    s = img_ref[:, 0:8, :]
    for t in range(1, _TB // 8):
        s = s + img_ref[:, 8 * t:8 * (t + 1), :]
    acc_ref[...] += s

    @pl.when(g == pl.num_programs(0) - 1)
    def _epilogue():
        pooled = jnp.sum(acc_ref[...], axis=1) * (1.0 / _NTOK)
        q_img = lax.dot_general(
            pooled, wimg_ref[...], (((1,), (1,)), ((), ())),
            preferred_element_type=jnp.float32,
            precision=lax.Precision.HIGHEST) + bimg_ref[...]
        q_emo = lax.dot_general(
            emo_ref[...], wemo_ref[...], (((1,), (1,)), ((), ())),
            preferred_element_type=jnp.float32,
            precision=lax.Precision.HIGHEST) + bemo_ref[...]
        q = jnp.concatenate([q_emo, q_img], axis=1)
        q_ref[...] = q - jnp.mean(q, axis=1, keepdims=True)


def _select_body(style_ref, q_ref, idx_ref, rows_ref, sacc_ref):
    g = pl.program_id(0)
    x = style_ref[...]
    cs = jnp.sum(x, axis=1)
    norm = jnp.sqrt(jnp.sum(cs * cs, axis=1, keepdims=True))
    denom = jnp.maximum(norm, _TOK * 1e-6)
    sblk = lax.dot_general(
        cs, q_ref[...], (((1,), (1,)), ((), ())),
        preferred_element_type=jnp.float32,
        precision=lax.Precision.HIGHEST)
    sacc_ref[g] = sblk / denom

    @pl.when(g == pl.num_programs(0) - 1)
    def _argmax():
        _argmax_epilogue(sacc_ref, idx_ref, rows_ref)


def _argmax_epilogue(sacc_ref, idx_ref, rows_ref):
    s = sacc_ref[...].reshape(_K, _B)
    m = jnp.max(s, axis=0, keepdims=True)
    kiota = lax.broadcasted_iota(jnp.int32, (_K, _B), 0)
    cand = jnp.where(s == m, kiota, _K)
    idx = jnp.min(cand, axis=0, keepdims=True)
    idx_ref[...] = idx
    eye = (lax.broadcasted_iota(jnp.int32, (_B, _B), 0) ==
           lax.broadcasted_iota(jnp.int32, (_B, _B), 1)).astype(jnp.float32)
    idx_col = lax.dot_general(
        eye, idx.astype(jnp.float32), (((1,), (1,)), ((), ())),
        preferred_element_type=jnp.float32,
        precision=lax.Precision.HIGHEST)
    tiota = lax.broadcasted_iota(jnp.int32, (_B, _TOK), 1)
    rows_ref[...] = idx_col.astype(jnp.int32) * _TOK + tiota


@functools.lru_cache(maxsize=1)
def _make_sc_gather():
    NC, NS = 2, 16
    NW = NC * NS
    NROWS = _B * _TOK
    CH = 16
    rows_per_w = NROWS // NW
    n_ch = rows_per_w // CH

    @functools.partial(
        pl.kernel,
        out_type=jax.ShapeDtypeStruct((NROWS, _EMB), jnp.float32),
        mesh=plsc.VectorSubcoreMesh(core_axis_name="c", subcore_axis_name="s"),
        scratch_types=[
            pltpu.VMEM((rows_per_w,), jnp.int32),
            pltpu.VMEM((3, CH, _EMB), jnp.float32),
            pltpu.SemaphoreType.DMA,
            pltpu.SemaphoreType.DMA,
            pltpu.SemaphoreType.DMA,
            pltpu.SemaphoreType.DMA,
            pltpu.SemaphoreType.DMA,
            pltpu.SemaphoreType.DMA,
        ],
    )
    def gather_kernel(table_hbm, rows_hbm, out_hbm, idxall, buf, g0, g1, g2,
                      w0, w1, w2):
        gsem = (g0, g1, g2)
        wsem = (w0, w1, w2)
        wid = lax.axis_index("s") * NC + lax.axis_index("c")
        wbase = wid * rows_per_w
        pltpu.sync_copy(rows_hbm.at[pl.ds(wbase, rows_per_w)], idxall)

        def gather(j):
            return pltpu.async_copy(
                table_hbm.at[idxall.at[pl.ds(j * CH, CH)]],
                buf.at[j % 3], gsem[j % 3])

        def scatter(j):
            return pltpu.async_copy(
                buf.at[j % 3], out_hbm.at[pl.ds(wbase + j * CH, CH)],
                wsem[j % 3])

        gh = [None] * n_ch
        sh = [None] * n_ch
        gh[0] = gather(0)
        gh[1] = gather(1)
        for j in range(n_ch):
            if j + 2 < n_ch:
                if j >= 1:
                    sh[j - 1].wait()
                gh[j + 2] = gather(j + 2)
            gh[j].wait()
            sh[j] = scatter(j)
        for j in range(max(0, n_ch - 3), n_ch):
            sh[j].wait()

    return gather_kernel


def kernel(emo_vec, image_tokens, W_emo, b_emo, W_img, b_img, style_dict):
    qc = pl.pallas_call(
        _query_body,
        grid=(_NTOK // _TB,),
        in_specs=[
            pl.BlockSpec((_B, _TB, _IMG_DIM), lambda g: (0, g, 0)),
            pl.BlockSpec((_B, _IN_DIM), lambda g: (0, 0)),
            pl.BlockSpec((_HALF, _IN_DIM), lambda g: (0, 0)),
            pl.BlockSpec((1, _HALF), lambda g: (0, 0)),
            pl.BlockSpec((_HALF, _IMG_DIM), lambda g: (0, 0)),
            pl.BlockSpec((1, _HALF), lambda g: (0, 0)),
        ],
        out_specs=pl.BlockSpec((_B, _EMB), lambda g: (0, 0)),
        out_shape=jax.ShapeDtypeStruct((_B, _EMB), jnp.float32),
        scratch_shapes=[pltpu.VMEM((_B, 8, _IMG_DIM), jnp.float32)],
    )(image_tokens, emo_vec, W_emo, b_emo.reshape(1, _HALF), W_img,
      b_img.reshape(1, _HALF))

    idx2d, rows2d = pl.pallas_call(
        _select_body,
        grid=(_K // _KB,),
        in_specs=[
            pl.BlockSpec((_KB, _TOK, _EMB), lambda g: (g, 0, 0)),
            pl.BlockSpec((_B, _EMB), lambda g: (0, 0)),
        ],
        out_specs=[pl.BlockSpec((1, _B), lambda g: (0, 0)),
                   pl.BlockSpec((_B, _TOK), lambda g: (0, 0))],
        out_shape=[jax.ShapeDtypeStruct((1, _B), jnp.int32),
                   jax.ShapeDtypeStruct((_B, _TOK), jnp.int32)],
        scratch_shapes=[pltpu.VMEM((_K // _KB, _KB, _B), jnp.float32)],
    )(style_dict, qc)

    indices = idx2d.reshape(_B)
    table = style_dict.reshape(_K * _TOK, _EMB)
    rows = rows2d.reshape(_B * _TOK)
    style = _make_sc_gather()(table, rows).reshape(_B, _TOK, _EMB)
    return style, indices


# speedup vs baseline: 1.9426x; 1.0724x over previous
import functools

import jax
import jax.numpy as jnp
from jax import lax
from jax.experimental import pallas as pl
from jax.experimental.pallas import tpu as pltpu
from jax.experimental.pallas import tpu_sc as plsc

_B = 64
_IN_DIM = 8
_IMG_DIM = 1152
_TOK = 64
_EMB = 2048
_K = 512
_NTOK = 576
_HALF = 1024

_TB = 48
_KB = 16


_NI = _NTOK // _TB
_NS = _K // _KB


def _fused_body(img_ref, style_ref, emo_ref, wemo_ref, bemo_ref, wimg_ref,
                bimg_ref, idx_ref, rows_ref, acc_ref, q_ref, cn_ref):
    g = pl.program_id(0)

    @pl.when(g == 0)
    def _init():
        acc_ref[...] = jnp.zeros_like(acc_ref)

    @pl.when(g < _NI)
    def _image_phase():
        s = img_ref[:, 0:8, :]
        for t in range(1, _TB // 8):
            s = s + img_ref[:, 8 * t:8 * (t + 1), :]
        acc_ref[...] += s

    @pl.when(g == _NI - 1)
    def _query():
        pooled = jnp.sum(acc_ref[...], axis=1) * (1.0 / _NTOK)
        q_img = lax.dot_general(
            pooled, wimg_ref[...], (((1,), (1,)), ((), ())),
            preferred_element_type=jnp.float32,
            precision=lax.Precision.HIGHEST) + bimg_ref[...]
        q_emo = lax.dot_general(
            emo_ref[...], wemo_ref[...], (((1,), (1,)), ((), ())),
            preferred_element_type=jnp.float32,
            precision=lax.Precision.HIGHEST) + bemo_ref[...]
        q = jnp.concatenate([q_emo, q_img], axis=1)
        q_ref[...] = q - jnp.mean(q, axis=1, keepdims=True)

    @pl.when(g >= _NI)
    def _style_phase():
        x = style_ref[...]
        cs = jnp.sum(x, axis=1)
        norm = jnp.sqrt(jnp.sum(cs * cs, axis=1, keepdims=True))
        cn_ref[g - _NI] = cs / jnp.maximum(norm, _TOK * 1e-6)

    @pl.when(g == _NI + _NS - 1)
    def _argmax():
        _argmax_epilogue(cn_ref, q_ref, idx_ref, rows_ref)


def _argmax_epilogue(cn_ref, q_ref, idx_ref, rows_ref):
    s = lax.dot_general(
        cn_ref[...].reshape(_K, _EMB), q_ref[...], (((1,), (1,)), ((), ())),
        preferred_element_type=jnp.float32,
        precision=lax.Precision.HIGHEST)
    m = jnp.max(s, axis=0, keepdims=True)
    kiota = lax.broadcasted_iota(jnp.int32, (_K, _B), 0)
    cand = jnp.where(s == m, kiota, _K)
    idx = jnp.min(cand, axis=0, keepdims=True)
    idx_ref[...] = idx
    eye = (lax.broadcasted_iota(jnp.int32, (_B, _B), 0) ==
           lax.broadcasted_iota(jnp.int32, (_B, _B), 1)).astype(jnp.float32)
    idx_col = lax.dot_general(
        eye, idx.astype(jnp.float32), (((1,), (1,)), ((), ())),
        preferred_element_type=jnp.float32,
        precision=lax.Precision.HIGHEST)
    tiota = lax.broadcasted_iota(jnp.int32, (_B, _TOK), 1)
    rows_ref[...] = idx_col.astype(jnp.int32) * _TOK + tiota


@functools.lru_cache(maxsize=1)
def _make_sc_gather():
    NC, NS = 2, 16
    NW = NC * NS
    NROWS = _B * _TOK
    CH = 16
    rows_per_w = NROWS // NW
    n_ch = rows_per_w // CH

    @functools.partial(
        pl.kernel,
        out_type=jax.ShapeDtypeStruct((NROWS, _EMB), jnp.float32),
        mesh=plsc.VectorSubcoreMesh(core_axis_name="c", subcore_axis_name="s"),
        scratch_types=[
            pltpu.VMEM((rows_per_w,), jnp.int32),
            pltpu.VMEM((3, CH, _EMB), jnp.float32),
            pltpu.SemaphoreType.DMA,
            pltpu.SemaphoreType.DMA,
            pltpu.SemaphoreType.DMA,
            pltpu.SemaphoreType.DMA,
            pltpu.SemaphoreType.DMA,
            pltpu.SemaphoreType.DMA,
        ],
    )
    def gather_kernel(table_hbm, rows_hbm, out_hbm, idxall, buf, g0, g1, g2,
                      w0, w1, w2):
        gsem = (g0, g1, g2)
        wsem = (w0, w1, w2)
        wid = lax.axis_index("s") * NC + lax.axis_index("c")
        wbase = wid * rows_per_w
        pltpu.sync_copy(rows_hbm.at[pl.ds(wbase, rows_per_w)], idxall)

        def gather(j):
            return pltpu.async_copy(
                table_hbm.at[idxall.at[pl.ds(j * CH, CH)]],
                buf.at[j % 3], gsem[j % 3])

        def scatter(j):
            return pltpu.async_copy(
                buf.at[j % 3], out_hbm.at[pl.ds(wbase + j * CH, CH)],
                wsem[j % 3])

        gh = [None] * n_ch
        sh = [None] * n_ch
        gh[0] = gather(0)
        gh[1] = gather(1)
        for j in range(n_ch):
            if j + 2 < n_ch:
                if j >= 1:
                    sh[j - 1].wait()
                gh[j + 2] = gather(j + 2)
            gh[j].wait()
            sh[j] = scatter(j)
        for j in range(max(0, n_ch - 3), n_ch):
            sh[j].wait()

    return gather_kernel


def kernel(emo_vec, image_tokens, W_emo, b_emo, W_img, b_img, style_dict):
    idx2d, rows2d = pl.pallas_call(
        _fused_body,
        grid=(_NI + _NS,),
        in_specs=[
            pl.BlockSpec((_B, _TB, _IMG_DIM),
                         lambda g: (0, jnp.minimum(g, _NI - 1), 0)),
            pl.BlockSpec((_KB, _TOK, _EMB),
                         lambda g: (jnp.maximum(g - _NI, 0), 0, 0)),
            pl.BlockSpec((_B, _IN_DIM), lambda g: (0, 0)),
            pl.BlockSpec((_HALF, _IN_DIM), lambda g: (0, 0)),
            pl.BlockSpec((1, _HALF), lambda g: (0, 0)),
            pl.BlockSpec((_HALF, _IMG_DIM), lambda g: (0, 0)),
            pl.BlockSpec((1, _HALF), lambda g: (0, 0)),
        ],
        out_specs=[pl.BlockSpec((1, _B), lambda g: (0, 0)),
                   pl.BlockSpec((_B, _TOK), lambda g: (0, 0))],
        out_shape=[jax.ShapeDtypeStruct((1, _B), jnp.int32),
                   jax.ShapeDtypeStruct((_B, _TOK), jnp.int32)],
        scratch_shapes=[pltpu.VMEM((_B, 8, _IMG_DIM), jnp.float32),
                        pltpu.VMEM((_B, _EMB), jnp.float32),
                        pltpu.VMEM((_NS, _KB, _EMB), jnp.float32)],
    )(image_tokens, style_dict, emo_vec, W_emo, b_emo.reshape(1, _HALF),
      W_img, b_img.reshape(1, _HALF))

    indices = idx2d.reshape(_B)
    table = style_dict.reshape(_K * _TOK, _EMB)
    rows = rows2d.reshape(_B * _TOK)
    style = _make_sc_gather()(table, rows).reshape(_B, _TOK, _EMB)
    return style, indices
